# baseline (speedup 1.0000x reference)
import functools

import jax
import jax.numpy as jnp
from jax.experimental import pallas as pl
from jax.experimental.pallas import tpu as pltpu

_POOL_PAD = 128
_NEG = -3e38


def _sim_body(x_ref, pk_ref, out_ref, *, pool):
    x = x_ref[...]
    xm = jnp.mean(x, axis=1)
    ss = jnp.sum(xm * xm, axis=1, keepdims=True)
    xn = xm * jax.lax.rsqrt(jnp.maximum(ss, 1e-12))
    pk = pk_ref[...]
    ps = jnp.sum(pk * pk, axis=1, keepdims=True)
    pn = pk * jax.lax.rsqrt(jnp.maximum(ps, 1e-12))
    sim = jax.lax.dot_general(xn, pn, (((1,), (1,)), ((), ())),
                              preferred_element_type=jnp.float32)
    out_ref[:, :pool] = sim
    out_ref[:, pool:] = jnp.full((x.shape[0], _POOL_PAD - pool), _NEG,
                                 jnp.float32)


def _topk_hist_body(sim_ref, out_ref, *, top_k):
    sim = sim_ref[...]
    b = sim.shape[0]
    col = jax.lax.broadcasted_iota(jnp.int32, (b, _POOL_PAD), 1)
    hist2d = jnp.zeros((b, _POOL_PAD), jnp.int32)
    work = sim
    for _ in range(top_k):
        m = jnp.max(work, axis=1, keepdims=True)
        cand = jnp.where(work == m, col, jnp.int32(1 << 30))
        a = jnp.min(cand, axis=1, keepdims=True)
        pick = col == a
        hist2d = hist2d + pick.astype(jnp.int32)
        work = jnp.where(pick, _NEG, work)
    hist = jnp.sum(hist2d, axis=0, keepdims=True)
    colr = jax.lax.broadcasted_iota(jnp.int32, (1, _POOL_PAD), 1)
    key = hist * 256 + (255 - colr)
    for t in range(top_k):
        m = jnp.max(key)
        out_ref[t] = 255 - (m % 256)
        key = jnp.where(key == m, jnp.int32(-1), key)


def _gather_body(ids_ref, prompt_ref, out_ref, *, top_k, length):
    blk = out_ref.shape[0]
    for t in range(top_k):
        row = prompt_ref[pl.ds(ids_ref[t], 1)]
        out_ref[:, t * length:(t + 1) * length, :] = jnp.broadcast_to(
            row, (blk, length, row.shape[2]))


def kernel(x_embed, prompt, prompt_key):
    b, s, d = x_embed.shape
    pool, length, _ = prompt.shape
    top_k = 8

    bblk = 16
    sim = pl.pallas_call(
        functools.partial(_sim_body, pool=pool),
        grid=(b // bblk,),
        in_specs=[
            pl.BlockSpec((bblk, s, d), lambda i: (i, 0, 0)),
            pl.BlockSpec((pool, d), lambda i: (0, 0)),
        ],
        out_specs=pl.BlockSpec((bblk, _POOL_PAD), lambda i: (i, 0)),
        out_shape=jax.ShapeDtypeStruct((b, _POOL_PAD), jnp.float32),
    )(x_embed, prompt_key)

    ids = pl.pallas_call(
        functools.partial(_topk_hist_body, top_k=top_k),
        in_specs=[pl.BlockSpec((b, _POOL_PAD), lambda: (0, 0))],
        out_specs=pl.BlockSpec(memory_space=pltpu.SMEM),
        out_shape=jax.ShapeDtypeStruct((top_k,), jnp.int32),
    )(sim)

    gblk = 32
    out = pl.pallas_call(
        functools.partial(_gather_body, top_k=top_k, length=length),
        grid=(b // gblk,),
        in_specs=[
            pl.BlockSpec(memory_space=pltpu.SMEM),
            pl.BlockSpec((pool, length, d), lambda i: (0, 0, 0)),
        ],
        out_specs=pl.BlockSpec((gblk, top_k * length, d), lambda i: (i, 0, 0)),
        out_shape=jax.ShapeDtypeStruct((b, top_k * length, d), jnp.float32),
    )(ids, prompt)
    return out
